# Initial kernel scaffold; baseline (speedup 1.0000x reference)
import jax
import jax.numpy as jnp
from jax.experimental import pallas as pl


def kernel(x, gamma, beta):
    raise NotImplementedError("write your pallas kernel here")


# speedup vs baseline: 21.1310x; 21.1310x over previous
import functools

import jax
import jax.numpy as jnp
import numpy as np
from jax.experimental import pallas as pl
from jax.experimental.pallas import tpu as pltpu

ALPHA_ = 0.99
EPS_ = 1e-6
EMA_INIT_ = 1e-4
L_ = 256


def _ema_norm_kernel(x_ref, a_ref, pow_ref, gamma_ref, beta_ref, o_ref,
                     carry_ref, *, L):
    chunk = pl.program_id(1)

    @pl.when(chunk == 0)
    def _():
        carry_ref[...] = jnp.full_like(carry_ref, EMA_INIT_)

    x = x_ref[0]
    b = (1.0 - ALPHA_) * x * x
    acc = jnp.dot(a_ref[...], b, preferred_element_type=jnp.float32)
    powv = pow_ref[...]
    ema = acc + powv * carry_ref[...]
    carry_ref[...] = ema[L - 1:L, :]

    ln_a = float(np.log(ALPHA_))
    s = jnp.exp(chunk.astype(jnp.float32) * (L * ln_a))
    denom = (1.0 + EPS_) - s * powv
    g = jnp.sqrt(ema / denom + EPS_)
    m = jnp.mean(g, axis=-1, keepdims=True)
    n = g / (m + EPS_)
    o_ref[0] = gamma_ref[...] * (x * n) + beta_ref[...] + x


@jax.jit
def kernel(x, gamma, beta):
    B, T, C = x.shape
    L = L_
    num_chunks = T // L

    i = np.arange(L)
    amat = np.where(i[:, None] >= i[None, :],
                    ALPHA_ ** (i[:, None] - i[None, :]), 0.0)
    amat = jnp.asarray(amat, dtype=jnp.float32)
    powv = jnp.asarray(
        np.broadcast_to((ALPHA_ ** (i + 1))[:, None], (L, C)).copy(),
        dtype=jnp.float32)

    grid = (B, num_chunks)
    out = pl.pallas_call(
        functools.partial(_ema_norm_kernel, L=L),
        grid=grid,
        in_specs=[
            pl.BlockSpec((1, L, C), lambda b, t: (b, t, 0)),
            pl.BlockSpec((L, L), lambda b, t: (0, 0)),
            pl.BlockSpec((L, C), lambda b, t: (0, 0)),
            pl.BlockSpec((1, C), lambda b, t: (0, 0)),
            pl.BlockSpec((1, C), lambda b, t: (0, 0)),
        ],
        out_specs=pl.BlockSpec((1, L, C), lambda b, t: (b, t, 0)),
        out_shape=jax.ShapeDtypeStruct((B, T, C), x.dtype),
        scratch_shapes=[pltpu.VMEM((1, C), jnp.float32)],
        compiler_params=pltpu.CompilerParams(
            dimension_semantics=("parallel", "arbitrary"),
        ),
    )(x, amat, powv, gamma, beta)
    return out


# speedup vs baseline: 38.9752x; 1.8445x over previous
import functools

import jax
import jax.numpy as jnp
import numpy as np
from jax.experimental import pallas as pl
from jax.experimental.pallas import tpu as pltpu

ALPHA_ = 0.99
EPS_ = 1e-6
EMA_INIT_ = 1e-4
L_ = 256
G_ = 4


def _ema_norm_kernel(x_ref, a_ref, pow_ref, gamma_ref, beta_ref, o_ref,
                     carry_ref, *, L, G):
    chunk = pl.program_id(1)

    @pl.when(chunk == 0)
    def _():
        carry_ref[...] = jnp.full_like(carry_ref, EMA_INIT_)

    amat = a_ref[...]
    powv = pow_ref[...]
    ln_a = float(np.log(ALPHA_))
    s = jnp.exp(chunk.astype(jnp.float32) * (L * ln_a))
    denom = (1.0 + EPS_) - s * powv
    gamma = gamma_ref[...]
    beta = beta_ref[...]

    for g in range(G):
        x = x_ref[g]
        acc = jnp.dot(amat, x * x, preferred_element_type=jnp.float32)
        ema = acc + powv * carry_ref[g:g + 1]
        carry_ref[g:g + 1] = ema[L - 1:L, :]
        gn = jnp.sqrt(ema / denom + EPS_)
        m = jnp.mean(gn, axis=-1, keepdims=True)
        n = gn / (m + EPS_)
        o_ref[g] = gamma * (x * n) + beta + x


@jax.jit
def kernel(x, gamma, beta):
    B, T, C = x.shape
    L = L_
    G = G_
    num_chunks = T // L

    i = np.arange(L)
    amat = np.where(i[:, None] >= i[None, :],
                    (1.0 - ALPHA_) * ALPHA_ ** (i[:, None] - i[None, :]), 0.0)
    amat = jnp.asarray(amat, dtype=jnp.float32)
    powv = jnp.asarray(
        np.broadcast_to((ALPHA_ ** (i + 1))[:, None], (L, C)).copy(),
        dtype=jnp.float32)

    grid = (B // G, num_chunks)
    out = pl.pallas_call(
        functools.partial(_ema_norm_kernel, L=L, G=G),
        grid=grid,
        in_specs=[
            pl.BlockSpec((G, L, C), lambda b, t: (b, t, 0)),
            pl.BlockSpec((L, L), lambda b, t: (0, 0)),
            pl.BlockSpec((L, C), lambda b, t: (0, 0)),
            pl.BlockSpec((1, C), lambda b, t: (0, 0)),
            pl.BlockSpec((1, C), lambda b, t: (0, 0)),
        ],
        out_specs=pl.BlockSpec((G, L, C), lambda b, t: (b, t, 0)),
        out_shape=jax.ShapeDtypeStruct((B, T, C), x.dtype),
        scratch_shapes=[pltpu.VMEM((G, C), jnp.float32)],
        compiler_params=pltpu.CompilerParams(
            dimension_semantics=("parallel", "arbitrary"),
        ),
    )(x, amat, powv, gamma, beta)
    return out


# speedup vs baseline: 46.1058x; 1.1830x over previous
import functools

import jax
import jax.numpy as jnp
import numpy as np
from jax.experimental import pallas as pl
from jax.experimental.pallas import tpu as pltpu

ALPHA_ = 0.99
EPS_ = 1e-6
EMA_INIT_ = 1e-4
L_ = 256
G_ = 8


def _ema_norm_kernel(x_ref, a_ref, pow_ref, gamma_ref, beta_ref, o_ref,
                     carry_ref, *, L, G):
    chunk = pl.program_id(1)

    @pl.when(chunk == 0)
    def _():
        carry_ref[...] = jnp.full_like(carry_ref, EMA_INIT_)

    amat = a_ref[...]
    powv = pow_ref[...]
    ln_a = float(np.log(ALPHA_))
    s = jnp.exp(chunk.astype(jnp.float32) * (L * ln_a))
    denom = (1.0 + EPS_) - s * powv
    gamma = gamma_ref[...]
    beta = beta_ref[...]

    for g in range(G):
        x = x_ref[g]
        acc = jnp.dot(amat, x * x, preferred_element_type=jnp.float32)
        ema = acc + powv * carry_ref[g:g + 1]
        carry_ref[g:g + 1] = ema[L - 1:L, :]
        gn = jnp.sqrt(ema / denom + EPS_)
        m = jnp.mean(gn, axis=-1, keepdims=True)
        n = gn / (m + EPS_)
        o_ref[g] = gamma * (x * n) + beta + x


@jax.jit
def kernel(x, gamma, beta):
    B, T, C = x.shape
    L = L_
    G = G_
    num_chunks = T // L

    i = np.arange(L)
    amat = np.where(i[:, None] >= i[None, :],
                    (1.0 - ALPHA_) * ALPHA_ ** (i[:, None] - i[None, :]), 0.0)
    amat = jnp.asarray(amat, dtype=jnp.float32)
    powv = jnp.asarray(
        np.broadcast_to((ALPHA_ ** (i + 1))[:, None], (L, C)).copy(),
        dtype=jnp.float32)

    grid = (B // G, num_chunks)
    out = pl.pallas_call(
        functools.partial(_ema_norm_kernel, L=L, G=G),
        grid=grid,
        in_specs=[
            pl.BlockSpec((G, L, C), lambda b, t: (b, t, 0)),
            pl.BlockSpec((L, L), lambda b, t: (0, 0)),
            pl.BlockSpec((L, C), lambda b, t: (0, 0)),
            pl.BlockSpec((1, C), lambda b, t: (0, 0)),
            pl.BlockSpec((1, C), lambda b, t: (0, 0)),
        ],
        out_specs=pl.BlockSpec((G, L, C), lambda b, t: (b, t, 0)),
        out_shape=jax.ShapeDtypeStruct((B, T, C), x.dtype),
        scratch_shapes=[pltpu.VMEM((G, C), jnp.float32)],
        compiler_params=pltpu.CompilerParams(
            dimension_semantics=("parallel", "arbitrary"),
        ),
    )(x, amat, powv, gamma, beta)
    return out


# speedup vs baseline: 50.4700x; 1.0947x over previous
import functools

import jax
import jax.numpy as jnp
import numpy as np
from jax.experimental import pallas as pl
from jax.experimental.pallas import tpu as pltpu

ALPHA_ = 0.99
EPS_ = 1e-6
EMA_INIT_ = 1e-4
L_ = 512
G_ = 8


def _ema_norm_kernel(x_ref, a_ref, pow_ref, gamma_ref, beta_ref, o_ref,
                     carry_ref, *, L, G):
    chunk = pl.program_id(1)

    @pl.when(chunk == 0)
    def _():
        carry_ref[...] = jnp.full_like(carry_ref, EMA_INIT_)

    amat = a_ref[...]
    powv = pow_ref[...]
    ln_a = float(np.log(ALPHA_))
    s = jnp.exp(chunk.astype(jnp.float32) * (L * ln_a))
    denom = (1.0 + EPS_) - s * powv
    gamma = gamma_ref[...]
    beta = beta_ref[...]

    for g in range(G):
        x = x_ref[g]
        acc = jnp.dot(amat, x * x, preferred_element_type=jnp.float32)
        ema = acc + powv * carry_ref[g:g + 1]
        carry_ref[g:g + 1] = ema[L - 1:L, :]
        gn = jnp.sqrt(ema / denom + EPS_)
        m = jnp.mean(gn, axis=-1, keepdims=True)
        n = gn / (m + EPS_)
        o_ref[g] = gamma * (x * n) + beta + x


@jax.jit
def kernel(x, gamma, beta):
    B, T, C = x.shape
    L = L_
    G = G_
    num_chunks = T // L

    i = np.arange(L)
    amat = np.where(i[:, None] >= i[None, :],
                    (1.0 - ALPHA_) * ALPHA_ ** (i[:, None] - i[None, :]), 0.0)
    amat = jnp.asarray(amat, dtype=jnp.float32)
    powv = jnp.asarray(
        np.broadcast_to((ALPHA_ ** (i + 1))[:, None], (L, C)).copy(),
        dtype=jnp.float32)

    grid = (B // G, num_chunks)
    out = pl.pallas_call(
        functools.partial(_ema_norm_kernel, L=L, G=G),
        grid=grid,
        in_specs=[
            pl.BlockSpec((G, L, C), lambda b, t: (b, t, 0)),
            pl.BlockSpec((L, L), lambda b, t: (0, 0)),
            pl.BlockSpec((L, C), lambda b, t: (0, 0)),
            pl.BlockSpec((1, C), lambda b, t: (0, 0)),
            pl.BlockSpec((1, C), lambda b, t: (0, 0)),
        ],
        out_specs=pl.BlockSpec((G, L, C), lambda b, t: (b, t, 0)),
        out_shape=jax.ShapeDtypeStruct((B, T, C), x.dtype),
        scratch_shapes=[pltpu.VMEM((G, C), jnp.float32)],
        compiler_params=pltpu.CompilerParams(
            dimension_semantics=("parallel", "arbitrary"),
        ),
    )(x, amat, powv, gamma, beta)
    return out


# speedup vs baseline: 55.5348x; 1.1004x over previous
import functools

import jax
import jax.numpy as jnp
import numpy as np
from jax.experimental import pallas as pl
from jax.experimental.pallas import tpu as pltpu

ALPHA_ = 0.99
EPS_ = 1e-6
EMA_INIT_ = 1e-4
L_ = 512
G_ = 8


def _ema_norm_kernel(x_ref, a_ref, pow_ref, gamma_ref, beta_ref, o_ref,
                     carry_ref, *, L, G):
    chunk = pl.program_id(1)

    @pl.when(chunk == 0)
    def _():
        carry_ref[...] = jnp.full_like(carry_ref, EMA_INIT_)

    amat = a_ref[...]
    powv = pow_ref[...]
    ln_a = float(np.log(ALPHA_))
    s = jnp.exp(chunk.astype(jnp.float32) * (L * ln_a))
    denom = (1.0 + EPS_) - s * powv
    gamma = gamma_ref[...]
    beta = beta_ref[...]

    for g in range(G):
        x = x_ref[g]
        acc = jnp.dot(amat, x * x, preferred_element_type=jnp.float32)
        ema = acc + powv * carry_ref[g:g + 1]
        carry_ref[g:g + 1] = ema[L - 1:L, :]
        v = ema / denom + EPS_
        gn = v * jax.lax.rsqrt(v)
        m = jnp.mean(gn, axis=-1, keepdims=True)
        n = gn / (m + EPS_)
        o_ref[g] = gamma * (x * n) + beta + x


@jax.jit
def kernel(x, gamma, beta):
    B, T, C = x.shape
    L = L_
    G = G_
    num_chunks = T // L

    i = np.arange(L)
    amat = np.where(i[:, None] >= i[None, :],
                    (1.0 - ALPHA_) * ALPHA_ ** (i[:, None] - i[None, :]), 0.0)
    amat = jnp.asarray(amat, dtype=jnp.float32)
    powv = jnp.asarray(
        np.broadcast_to((ALPHA_ ** (i + 1))[:, None], (L, C)).copy(),
        dtype=jnp.float32)

    grid = (B // G, num_chunks)
    out = pl.pallas_call(
        functools.partial(_ema_norm_kernel, L=L, G=G),
        grid=grid,
        in_specs=[
            pl.BlockSpec((G, L, C), lambda b, t: (b, t, 0)),
            pl.BlockSpec((L, L), lambda b, t: (0, 0)),
            pl.BlockSpec((L, C), lambda b, t: (0, 0)),
            pl.BlockSpec((1, C), lambda b, t: (0, 0)),
            pl.BlockSpec((1, C), lambda b, t: (0, 0)),
        ],
        out_specs=pl.BlockSpec((G, L, C), lambda b, t: (b, t, 0)),
        out_shape=jax.ShapeDtypeStruct((B, T, C), x.dtype),
        scratch_shapes=[pltpu.VMEM((G, C), jnp.float32)],
        compiler_params=pltpu.CompilerParams(
            dimension_semantics=("parallel", "arbitrary"),
        ),
    )(x, amat, powv, gamma, beta)
    return out
